# Initial kernel scaffold; baseline (speedup 1.0000x reference)
import jax
import jax.numpy as jnp
from jax.experimental import pallas as pl


def kernel(x, edge_index, W, b):
    raise NotImplementedError("write your pallas kernel here")


# speedup vs baseline: 8.2595x; 8.2595x over previous
import functools

import jax
import jax.numpy as jnp
from jax import lax
from jax.experimental import pallas as pl
from jax.experimental.pallas import tpu as pltpu
from jax.experimental.pallas import tpu_sc as plsc

N = 10000
E = 160000
CH = 256
H = CH // 2
P = 10240
EP = 163840
NC, NS, L = 2, 16, 16
B = 128
KA = EP // (NC * NS) // B
KC = EP // NS // B
RPT = P // NS
BR = 256

_mesh = plsc.VectorSubcoreMesh(
    core_axis_name="c", subcore_axis_name="s", num_cores=NC, num_subcores=NS
)


@functools.partial(
    pl.kernel,
    out_type=jax.ShapeDtypeStruct((NC, P, H), jnp.float32),
    mesh=_mesh,
    scratch_types=[
        pltpu.VMEM((KA, B), jnp.int32),
        pltpu.VMEM((B, H), jnp.float32),
        pltpu.VMEM_SHARED((P, H), jnp.float32),
    ],
)
def _deg_kernel(rows_hbm, ones_hbm, zeros_hbm, out_hbm, idx_v, ones_v, hist_sh):
    cid = lax.axis_index("c")
    sid = lax.axis_index("s")
    wid = cid * NS + sid
    r0 = sid * RPT
    pltpu.sync_copy(rows_hbm.at[wid], idx_v)
    pltpu.sync_copy(ones_hbm, ones_v)
    pltpu.sync_copy(zeros_hbm, hist_sh.at[pl.ds(r0, RPT)])
    plsc.subcore_barrier()

    def body(bi, carry):
        pltpu.sync_copy(ones_v, hist_sh.at[idx_v.at[bi]], add=True)
        return carry

    lax.fori_loop(0, KA, body, 0)
    plsc.subcore_barrier()
    pltpu.sync_copy(hist_sh.at[pl.ds(r0, RPT)], out_hbm.at[cid, pl.ds(r0, RPT)])


def _mm_body(x_ref, w_ref, p_ref, y_ref):
    deg = p_ref[0, :, 0:1] + p_ref[1, :, 0:1] + 1.0
    d = 1.0 / jnp.sqrt(deg)
    z = jnp.dot(x_ref[...], w_ref[...], preferred_element_type=jnp.float32)
    y_ref[0] = z * d


_mm = pl.pallas_call(
    _mm_body,
    grid=(P // BR, NC),
    in_specs=[
        pl.BlockSpec((BR, CH), lambda i, c: (i, 0)),
        pl.BlockSpec((CH, H), lambda i, c: (0, c)),
        pl.BlockSpec((NC, BR, H), lambda i, c: (0, i, 0)),
    ],
    out_specs=pl.BlockSpec((1, BR, H), lambda i, c: (c, i, 0)),
    out_shape=jax.ShapeDtypeStruct((NC, P, H), jnp.float32),
)


@functools.partial(
    pl.kernel,
    out_type=jax.ShapeDtypeStruct((NC, P, H), jnp.float32),
    mesh=_mesh,
    scratch_types=[
        pltpu.VMEM((KC, B), jnp.int32),
        pltpu.VMEM((KC, B), jnp.int32),
        pltpu.VMEM((B, H), jnp.float32),
        pltpu.VMEM_SHARED((P, H), jnp.float32),
        pltpu.SemaphoreType.DMA,
    ],
)
def _prop_kernel(y_hbm, cols_hbm, rows_hbm, out_hbm, col_v, row_v, gbuf, acc_sh, sem):
    cid = lax.axis_index("c")
    sid = lax.axis_index("s")
    pltpu.sync_copy(cols_hbm.at[sid], col_v)
    pltpu.sync_copy(rows_hbm.at[sid], row_v)
    r0 = sid * RPT
    pltpu.sync_copy(y_hbm.at[cid, pl.ds(r0, RPT)], acc_sh.at[pl.ds(r0, RPT)])
    plsc.subcore_barrier()
    table = y_hbm.at[cid]

    def body(bi, carry):
        pltpu.async_copy(table.at[col_v.at[bi]], gbuf, sem).wait()
        pltpu.sync_copy(gbuf, acc_sh.at[row_v.at[bi]], add=True)
        return carry

    lax.fori_loop(0, KC, body, 0)
    plsc.subcore_barrier()
    pltpu.sync_copy(acc_sh.at[pl.ds(r0, RPT)], out_hbm.at[cid, pl.ds(r0, RPT)])


def _fin_body(a_ref, p_ref, b_ref, o_ref):
    deg = p_ref[0, :, 0:1] + p_ref[1, :, 0:1] + 1.0
    d = 1.0 / jnp.sqrt(deg)
    o_ref[...] = jnp.concatenate([a_ref[0] * d, a_ref[1] * d], axis=1) + b_ref[...]


_fin = pl.pallas_call(
    _fin_body,
    grid=(P // BR,),
    in_specs=[
        pl.BlockSpec((NC, BR, H), lambda i: (0, i, 0)),
        pl.BlockSpec((NC, BR, H), lambda i: (0, i, 0)),
        pl.BlockSpec((1, CH), lambda i: (0, 0)),
    ],
    out_specs=pl.BlockSpec((BR, CH), lambda i: (i, 0)),
    out_shape=jax.ShapeDtypeStruct((N, CH), jnp.float32),
)


def kernel(x, edge_index, W, b):
    ei = edge_index.astype(jnp.int32)
    pad = jnp.full((EP - E,), N, jnp.int32)
    row = jnp.concatenate([ei[0], pad])
    col = jnp.concatenate([ei[1], pad])
    rows_a = row.reshape(NC * NS, KA, B)
    rows_c = row.reshape(NS, KC, B)
    cols_c = col.reshape(NS, KC, B)
    ones_a = jnp.ones((B, H), jnp.float32)
    zeros_a = jnp.zeros((RPT, H), jnp.float32)
    x_pad = jnp.pad(x, ((0, P - N), (0, 0)))
    partial = _deg_kernel(rows_a, ones_a, zeros_a)
    y = _mm(x_pad, W, partial)
    acc = _prop_kernel(y, cols_c, rows_c)
    return _fin(acc, partial, b.reshape(1, CH))


# speedup vs baseline: 8.6673x; 1.0494x over previous
import functools

import jax
import jax.numpy as jnp
from jax import lax
from jax.experimental import pallas as pl
from jax.experimental.pallas import tpu as pltpu
from jax.experimental.pallas import tpu_sc as plsc

N = 10000
E = 160000
CH = 256
H = CH // 2
P = 10240
EP = 163840
NC, NS, L = 2, 16, 16
B = 128
KA = EP // (NC * NS) // B
KC = EP // NS // B
RPT = P // NS
BR = 256
GDEPTH = 1
NSLOT = GDEPTH + 1
RS = 4
SHIFT = 14

_mesh = plsc.VectorSubcoreMesh(
    core_axis_name="c", subcore_axis_name="s", num_cores=NC, num_subcores=NS
)


@functools.partial(
    pl.kernel,
    out_type=jax.ShapeDtypeStruct((NC, P, H), jnp.float32),
    mesh=_mesh,
    scratch_types=[
        pltpu.VMEM((KA, B), jnp.int32),
        pltpu.VMEM((B, H), jnp.float32),
        pltpu.VMEM_SHARED((P, H), jnp.float32),
        pltpu.SemaphoreType.DMA((2,)),
    ],
)
def _deg_kernel(rows_hbm, ones_hbm, zeros_hbm, out_hbm, idx_v, ones_v, hist_sh, sem):
    cid = lax.axis_index("c")
    sid = lax.axis_index("s")
    wid = cid * NS + sid
    r0 = sid * RPT
    pltpu.sync_copy(rows_hbm.at[wid], idx_v)
    pltpu.sync_copy(ones_hbm, ones_v)
    pltpu.sync_copy(zeros_hbm, hist_sh.at[pl.ds(r0, RPT)])
    plsc.subcore_barrier()

    def _scat(bi, slot):
        return pltpu.async_copy(
            ones_v, hist_sh.at[idx_v.at[bi]], sem.at[slot], add=True
        )

    _scat(0, 0)

    def body(bi, carry):
        slot = lax.rem(bi, 2)
        _scat(bi + 1, 1 - slot)
        pltpu.make_async_copy(
            ones_v, hist_sh.at[idx_v.at[bi]], sem.at[slot]
        ).wait()
        return carry

    lax.fori_loop(0, KA - 1, body, 0)
    pltpu.make_async_copy(
        ones_v, hist_sh.at[idx_v.at[KA - 1]], sem.at[(KA - 1) % 2]
    ).wait()
    plsc.subcore_barrier()
    pltpu.sync_copy(hist_sh.at[pl.ds(r0, RPT)], out_hbm.at[cid, pl.ds(r0, RPT)])


def _mm_body(x_ref, w_ref, p_ref, y_ref):
    deg = p_ref[0, :, 0:1] + p_ref[1, :, 0:1] + 1.0
    d = 1.0 / jnp.sqrt(deg)
    z = jnp.dot(x_ref[...], w_ref[...], preferred_element_type=jnp.float32)
    y_ref[0] = z * d


_mm = pl.pallas_call(
    _mm_body,
    grid=(P // BR, NC),
    in_specs=[
        pl.BlockSpec((BR, CH), lambda i, c: (i, 0)),
        pl.BlockSpec((CH, H), lambda i, c: (0, c)),
        pl.BlockSpec((NC, BR, H), lambda i, c: (0, i, 0)),
    ],
    out_specs=pl.BlockSpec((1, BR, H), lambda i, c: (c, i, 0)),
    out_shape=jax.ShapeDtypeStruct((NC, P, H), jnp.float32),
)


@functools.partial(
    pl.kernel,
    out_type=jax.ShapeDtypeStruct((NC, P, H), jnp.float32),
    mesh=_mesh,
    scratch_types=[
        pltpu.VMEM((KC, B), jnp.int32),
        pltpu.VMEM((RS, B), jnp.int32),
        pltpu.VMEM((RS, B), jnp.int32),
        pltpu.VMEM((NSLOT, B, H), jnp.float32),
        pltpu.VMEM_SHARED((P, H), jnp.float32),
        pltpu.SemaphoreType.DMA((NSLOT,)),
        pltpu.SemaphoreType.DMA((NSLOT,)),
    ],
)
def _prop_kernel(
    y_hbm, cr_hbm, out_hbm, cr_v, col_r, row_r, gbuf, acc_sh, gsem, ssem
):
    cid = lax.axis_index("c")
    sid = lax.axis_index("s")
    pltpu.sync_copy(cr_hbm.at[sid], cr_v)
    r0 = sid * RPT
    pltpu.sync_copy(y_hbm.at[cid, pl.ds(r0, RPT)], acc_sh.at[pl.ds(r0, RPT)])
    plsc.subcore_barrier()
    table = y_hbm.at[cid]

    def unpack(bi):
        rs = lax.rem(bi, RS)
        for j in range(B // L):
            v = cr_v[bi, pl.ds(j * L, L)]
            row_r[rs, pl.ds(j * L, L)] = jnp.right_shift(v, SHIFT)
            col_r[rs, pl.ds(j * L, L)] = jnp.bitwise_and(v, (1 << SHIFT) - 1)

    def start_gather(bi):
        slot = lax.rem(bi, NSLOT)
        pltpu.async_copy(
            table.at[col_r.at[lax.rem(bi, RS)]], gbuf.at[slot], gsem.at[slot]
        )

    def wait_gather(bi):
        slot = lax.rem(bi, NSLOT)
        pltpu.make_async_copy(
            table.at[col_r.at[lax.rem(bi, RS)]], gbuf.at[slot], gsem.at[slot]
        ).wait()

    def start_scatter(bi):
        slot = lax.rem(bi, NSLOT)
        pltpu.async_copy(
            gbuf.at[slot],
            acc_sh.at[row_r.at[lax.rem(bi, RS)]],
            ssem.at[slot],
            add=True,
        )

    def wait_scatter(bi):
        slot = lax.rem(bi, NSLOT)
        pltpu.make_async_copy(
            gbuf.at[slot], acc_sh.at[row_r.at[lax.rem(bi, RS)]], ssem.at[slot]
        ).wait()

    unpack(0)
    start_gather(0)

    def body(bi, carry):
        unpack(bi + 1)
        wait_gather(bi)
        start_scatter(bi)

        @pl.when(bi >= 1)
        def _():
            wait_scatter(bi - 1)

        start_gather(bi + 1)
        return carry

    lax.fori_loop(0, KC - 1, body, 0)
    wait_gather(KC - 1)
    start_scatter(KC - 1)
    wait_scatter(KC - 2)
    wait_scatter(KC - 1)
    plsc.subcore_barrier()
    pltpu.sync_copy(acc_sh.at[pl.ds(r0, RPT)], out_hbm.at[cid, pl.ds(r0, RPT)])


def _fin_body(a_ref, p_ref, b_ref, o_ref):
    deg = p_ref[0, :, 0:1] + p_ref[1, :, 0:1] + 1.0
    d = 1.0 / jnp.sqrt(deg)
    o_ref[...] = jnp.concatenate([a_ref[0] * d, a_ref[1] * d], axis=1) + b_ref[...]


_fin = pl.pallas_call(
    _fin_body,
    grid=(P // BR,),
    in_specs=[
        pl.BlockSpec((NC, BR, H), lambda i: (0, i, 0)),
        pl.BlockSpec((NC, BR, H), lambda i: (0, i, 0)),
        pl.BlockSpec((1, CH), lambda i: (0, 0)),
    ],
    out_specs=pl.BlockSpec((BR, CH), lambda i: (i, 0)),
    out_shape=jax.ShapeDtypeStruct((N, CH), jnp.float32),
)


def kernel(x, edge_index, W, b):
    ei = edge_index.astype(jnp.int32)
    pad = jnp.full((EP - E,), N, jnp.int32)
    row = jnp.concatenate([ei[0], pad])
    col = jnp.concatenate([ei[1], pad])
    rows_a = row.reshape(NC * NS, KA, B)
    cr = ((row << SHIFT) | col).reshape(NS, KC, B)
    ones_a = jnp.ones((B, H), jnp.float32)
    zeros_a = jnp.zeros((RPT, H), jnp.float32)
    x_pad = jnp.pad(x, ((0, P - N), (0, 0)))
    partial = _deg_kernel(rows_a, ones_a, zeros_a)
    y = _mm(x_pad, W, partial)
    acc = _prop_kernel(y, cr)
    return _fin(acc, partial, b.reshape(1, CH))


# speedup vs baseline: 9.0871x; 1.0484x over previous
import functools

import jax
import jax.numpy as jnp
from jax import lax
from jax.experimental import pallas as pl
from jax.experimental.pallas import tpu as pltpu
from jax.experimental.pallas import tpu_sc as plsc

N = 10000
E = 160000
CH = 256
H = CH // 2
P = 10240
EP = 163840
NC, NS, L = 2, 16, 16
B = 128
BC = 64
KA = EP // (NC * NS) // B
KC = EP // NS // BC
RPT = P // NS
BR = 256
GDEPTH = 2
NSLOT = GDEPTH + 1
RS = 8
SHIFT = 14

_mesh = plsc.VectorSubcoreMesh(
    core_axis_name="c", subcore_axis_name="s", num_cores=NC, num_subcores=NS
)


@functools.partial(
    pl.kernel,
    out_type=jax.ShapeDtypeStruct((NC, P, H), jnp.float32),
    mesh=_mesh,
    scratch_types=[
        pltpu.VMEM((KA, B), jnp.int32),
        pltpu.VMEM((B, H), jnp.float32),
        pltpu.VMEM_SHARED((P, H), jnp.float32),
        pltpu.SemaphoreType.DMA((2,)),
    ],
)
def _deg_kernel(rows_hbm, ones_hbm, zeros_hbm, out_hbm, idx_v, ones_v, hist_sh, sem):
    cid = lax.axis_index("c")
    sid = lax.axis_index("s")
    wid = cid * NS + sid
    r0 = sid * RPT
    pltpu.sync_copy(rows_hbm.at[wid], idx_v)
    pltpu.sync_copy(ones_hbm, ones_v)
    pltpu.sync_copy(zeros_hbm, hist_sh.at[pl.ds(r0, RPT)])
    plsc.subcore_barrier()

    def _scat(bi, slot):
        return pltpu.async_copy(
            ones_v, hist_sh.at[idx_v.at[bi]], sem.at[slot], add=True
        )

    _scat(0, 0)

    def body(bi, carry):
        slot = lax.rem(bi, 2)
        _scat(bi + 1, 1 - slot)
        pltpu.make_async_copy(
            ones_v, hist_sh.at[idx_v.at[bi]], sem.at[slot]
        ).wait()
        return carry

    lax.fori_loop(0, KA - 1, body, 0)
    pltpu.make_async_copy(
        ones_v, hist_sh.at[idx_v.at[KA - 1]], sem.at[(KA - 1) % 2]
    ).wait()
    plsc.subcore_barrier()
    pltpu.sync_copy(hist_sh.at[pl.ds(r0, RPT)], out_hbm.at[cid, pl.ds(r0, RPT)])


def _mm_body(x_ref, w_ref, p_ref, y_ref):
    deg = p_ref[0, :, 0:1] + p_ref[1, :, 0:1] + 1.0
    d = 1.0 / jnp.sqrt(deg)
    z = jnp.dot(x_ref[...], w_ref[...], preferred_element_type=jnp.float32)
    y_ref[0] = z * d


_mm = pl.pallas_call(
    _mm_body,
    grid=(P // BR, NC),
    in_specs=[
        pl.BlockSpec((BR, CH), lambda i, c: (i, 0)),
        pl.BlockSpec((CH, H), lambda i, c: (0, c)),
        pl.BlockSpec((NC, BR, H), lambda i, c: (0, i, 0)),
    ],
    out_specs=pl.BlockSpec((1, BR, H), lambda i, c: (c, i, 0)),
    out_shape=jax.ShapeDtypeStruct((NC, P, H), jnp.float32),
)


@functools.partial(
    pl.kernel,
    out_type=jax.ShapeDtypeStruct((NC, P, H), jnp.float32),
    mesh=_mesh,
    scratch_types=[
        pltpu.VMEM((KC, BC), jnp.int32),
        pltpu.VMEM((RS, BC), jnp.int32),
        pltpu.VMEM((RS, BC), jnp.int32),
        pltpu.VMEM((NSLOT, BC, H), jnp.float32),
        pltpu.VMEM_SHARED((P, H), jnp.float32),
        pltpu.SemaphoreType.DMA((NSLOT,)),
        pltpu.SemaphoreType.DMA((NSLOT,)),
    ],
)
def _prop_kernel(
    y_hbm, cr_hbm, out_hbm, cr_v, col_r, row_r, gbuf, acc_sh, gsem, ssem
):
    cid = lax.axis_index("c")
    sid = lax.axis_index("s")
    pltpu.sync_copy(cr_hbm.at[sid], cr_v)
    r0 = sid * RPT
    pltpu.sync_copy(y_hbm.at[cid, pl.ds(r0, RPT)], acc_sh.at[pl.ds(r0, RPT)])
    plsc.subcore_barrier()
    table = y_hbm.at[cid]

    def unpack(bi):
        rs = lax.rem(bi, RS)
        for j in range(BC // L):
            v = cr_v[bi, pl.ds(j * L, L)]
            row_r[rs, pl.ds(j * L, L)] = jnp.right_shift(v, SHIFT)
            col_r[rs, pl.ds(j * L, L)] = jnp.bitwise_and(v, (1 << SHIFT) - 1)

    def start_gather(bi):
        slot = lax.rem(bi, NSLOT)
        pltpu.async_copy(
            table.at[col_r.at[lax.rem(bi, RS)]], gbuf.at[slot], gsem.at[slot]
        )

    def wait_gather(bi):
        slot = lax.rem(bi, NSLOT)
        pltpu.make_async_copy(
            table.at[col_r.at[lax.rem(bi, RS)]], gbuf.at[slot], gsem.at[slot]
        ).wait()

    def start_scatter(bi):
        slot = lax.rem(bi, NSLOT)
        pltpu.async_copy(
            gbuf.at[slot],
            acc_sh.at[row_r.at[lax.rem(bi, RS)]],
            ssem.at[slot],
            add=True,
        )

    def wait_scatter(bi):
        slot = lax.rem(bi, NSLOT)
        pltpu.make_async_copy(
            gbuf.at[slot], acc_sh.at[row_r.at[lax.rem(bi, RS)]], ssem.at[slot]
        ).wait()

    for bi in range(GDEPTH):
        unpack(bi)
        start_gather(bi)

    def body(bi, carry):
        wait_gather(bi)
        start_scatter(bi)

        @pl.when(bi >= 1)
        def _():
            wait_scatter(bi - 1)

        @pl.when(bi + GDEPTH < KC)
        def _():
            unpack(bi + GDEPTH)
            start_gather(bi + GDEPTH)

        return carry

    lax.fori_loop(0, KC, body, 0)
    wait_scatter(KC - 1)
    plsc.subcore_barrier()
    pltpu.sync_copy(acc_sh.at[pl.ds(r0, RPT)], out_hbm.at[cid, pl.ds(r0, RPT)])


def _fin_body(a_ref, p_ref, b_ref, o_ref):
    deg = p_ref[0, :, 0:1] + p_ref[1, :, 0:1] + 1.0
    d = 1.0 / jnp.sqrt(deg)
    o_ref[...] = jnp.concatenate([a_ref[0] * d, a_ref[1] * d], axis=1) + b_ref[...]


_fin = pl.pallas_call(
    _fin_body,
    grid=(P // BR,),
    in_specs=[
        pl.BlockSpec((NC, BR, H), lambda i: (0, i, 0)),
        pl.BlockSpec((NC, BR, H), lambda i: (0, i, 0)),
        pl.BlockSpec((1, CH), lambda i: (0, 0)),
    ],
    out_specs=pl.BlockSpec((BR, CH), lambda i: (i, 0)),
    out_shape=jax.ShapeDtypeStruct((N, CH), jnp.float32),
)


def kernel(x, edge_index, W, b):
    ei = edge_index.astype(jnp.int32)
    pad = jnp.full((EP - E,), N, jnp.int32)
    row = jnp.concatenate([ei[0], pad])
    col = jnp.concatenate([ei[1], pad])
    rows_a = row.reshape(NC * NS, KA, B)
    cr = ((row << SHIFT) | col).reshape(NS, KC, BC)
    ones_a = jnp.ones((B, H), jnp.float32)
    zeros_a = jnp.zeros((RPT, H), jnp.float32)
    x_pad = jnp.pad(x, ((0, P - N), (0, 0)))
    partial = _deg_kernel(rows_a, ones_a, zeros_a)
    y = _mm(x_pad, W, partial)
    acc = _prop_kernel(y, cr)
    return _fin(acc, partial, b.reshape(1, CH))
